# Initial kernel scaffold; baseline (speedup 1.0000x reference)
import functools

import jax
import jax.numpy as jnp
from jax import lax
from jax.experimental import pallas as pl
from jax.experimental.pallas import tpu as pltpu
from jax.experimental.pallas import tpu_sc as plsc

N_NODES = 10000
N_EDGES = 160000
D_IN = 256
D_HID = 128
N_CLASSES = 64

NC = 2
NS = 16
N_PAD = 10240
NPT = N_PAD // NS
CW = 100
ROWS = N_EDGES // CW
RPT = ROWS // (NC * NS)

_mesh = plsc.VectorSubcoreMesh(core_axis_name="c", subcore_axis_name="s")


def _deg_body(src_hbm, dst_hbm, out_hbm, sidx, didx, buf, acc_out, acc_in):
    c = lax.axis_index("c")
    s = lax.axis_index("s")
    for q in range(NPT // 16):
        buf[pl.ds(q * 16, 16)] = jnp.zeros((16,), jnp.float32)
    pltpu.sync_copy(buf, acc_out.at[pl.ds(s * NPT, NPT)])
    pltpu.sync_copy(buf, acc_in.at[pl.ds(s * NPT, NPT)])
    for q in range(CW // 16 + 1):
        buf[pl.ds(q * 16, 16)] = jnp.ones((16,), jnp.float32)
    plsc.subcore_barrier()

    row0 = (c * NS + s) * RPT
    pltpu.sync_copy(src_hbm.at[pl.ds(row0, RPT), :], sidx)
    pltpu.sync_copy(dst_hbm.at[pl.ds(row0, RPT), :], didx)

    def body(j, carry):
        pltpu.sync_copy(buf.at[pl.ds(0, CW)], acc_out.at[sidx.at[j]], add=True)
        pltpu.sync_copy(buf.at[pl.ds(0, CW)], acc_in.at[didx.at[j]], add=True)
        return carry

    lax.fori_loop(0, RPT, body, 0)
    plsc.subcore_barrier()
    pltpu.sync_copy(acc_out.at[pl.ds(s * NPT, NPT)], out_hbm.at[c, 0, pl.ds(s * NPT, NPT)])
    pltpu.sync_copy(acc_in.at[pl.ds(s * NPT, NPT)], out_hbm.at[c, 1, pl.ds(s * NPT, NPT)])


_deg_call = pl.kernel(
    _deg_body,
    out_type=jax.ShapeDtypeStruct((NC, 2, N_PAD), jnp.float32),
    mesh=_mesh,
    scratch_types=[
        pltpu.VMEM((RPT, CW), jnp.int32),
        pltpu.VMEM((RPT, CW), jnp.int32),
        pltpu.VMEM((NPT,), jnp.float32),
        pltpu.VMEM_SHARED((N_PAD,), jnp.float32),
        pltpu.VMEM_SHARED((N_PAD,), jnp.float32),
    ],
)


def _make_agg(D):
    def _agg_body(y_hbm, src_hbm, dst_hbm, out_hbm, sidx, didx, rows_v, zbuf, acc, sem):
        c = lax.axis_index("c")
        s = lax.axis_index("s")
        for r in range(16):
            for q in range(D // 16):
                zbuf[r, pl.ds(q * 16, 16)] = jnp.zeros((16,), jnp.float32)

        def zloop(k, carry):
            pltpu.sync_copy(zbuf, acc.at[pl.ds(s * NPT + k * 16, 16), :])
            return carry

        lax.fori_loop(0, NPT // 16, zloop, 0)
        plsc.subcore_barrier()

        row0 = (c * NS + s) * RPT
        pltpu.sync_copy(src_hbm.at[pl.ds(row0, RPT), :], sidx)
        pltpu.sync_copy(dst_hbm.at[pl.ds(row0, RPT), :], didx)

        def body(j, carry):
            pltpu.async_copy(y_hbm.at[sidx.at[j]], rows_v, sem).wait()
            pltpu.sync_copy(rows_v, acc.at[didx.at[j]], add=True)
            return carry

        lax.fori_loop(0, RPT, body, 0)
        plsc.subcore_barrier()
        pltpu.sync_copy(acc.at[pl.ds(s * NPT, NPT), :],
                        out_hbm.at[c, pl.ds(s * NPT, NPT), :])

    return pl.kernel(
        _agg_body,
        out_type=jax.ShapeDtypeStruct((NC, N_PAD, D), jnp.float32),
        mesh=_mesh,
        scratch_types=[
            pltpu.VMEM((RPT, CW), jnp.int32),
            pltpu.VMEM((RPT, CW), jnp.int32),
            pltpu.VMEM((CW, D), jnp.float32),
            pltpu.VMEM((16, D), jnp.float32),
            pltpu.VMEM_SHARED((N_PAD, D), jnp.float32),
            pltpu.SemaphoreType.DMA,
        ],
    )


_agg_hid = _make_agg(D_HID)
_agg_out = _make_agg(N_CLASSES)


def _norm(d):
    return jnp.where(d > 0.0, lax.rsqrt(jnp.maximum(d, 1.0)), 0.0)


def _y1_body(x_ref, w_ref, dp_ref, o_ref):
    d_out = dp_ref[0, 0, :N_NODES] + dp_ref[1, 0, :N_NODES]
    ns = _norm(d_out)
    z = jnp.dot(x_ref[...], w_ref[...], preferred_element_type=jnp.float32)
    o_ref[...] = z * ns[:, None]


def _y2_body(a_ref, dp_ref, b1_ref, w_ref, o_ref):
    a = a_ref[0, :N_NODES, :] + a_ref[1, :N_NODES, :]
    nd = _norm(dp_ref[0, 1, :N_NODES] + dp_ref[1, 1, :N_NODES])
    ns = _norm(dp_ref[0, 0, :N_NODES] + dp_ref[1, 0, :N_NODES])
    h = jnp.maximum(a * nd[:, None] + b1_ref[...][None, :], 0.0)
    o_ref[...] = jnp.dot(h * ns[:, None], w_ref[...],
                         preferred_element_type=jnp.float32)


def _out_body(a_ref, dp_ref, b2_ref, o_ref):
    a = a_ref[0, :N_NODES, :] + a_ref[1, :N_NODES, :]
    nd = _norm(dp_ref[0, 1, :N_NODES] + dp_ref[1, 1, :N_NODES])
    o_ref[...] = a * nd[:, None] + b2_ref[...][None, :]


def _tc_call(body, out_shape):
    return pl.pallas_call(body, out_shape=jax.ShapeDtypeStruct(out_shape, jnp.float32))


@jax.jit
def kernel(features, edge_index, W1, b1, W2, b2):
    src = edge_index[0].astype(jnp.int32).reshape(ROWS, CW)
    dst = edge_index[1].astype(jnp.int32).reshape(ROWS, CW)

    dp = _deg_call(src, dst)
    y1 = _tc_call(_y1_body, (N_NODES, D_HID))(features, W1, dp)
    p1 = _agg_hid(y1, src, dst)
    y2 = _tc_call(_y2_body, (N_NODES, N_CLASSES))(p1, dp, b1, W2)
    p2 = _agg_out(y2, src, dst)
    out = _tc_call(_out_body, (N_NODES, N_CLASSES))(p2, dp, b2)
    return out


# speedup vs baseline: 7.4750x; 7.4750x over previous
import functools

import jax
import jax.numpy as jnp
from jax import lax
from jax.experimental import pallas as pl
from jax.experimental.pallas import tpu as pltpu
from jax.experimental.pallas import tpu_sc as plsc

N_NODES = 10000
N_EDGES = 160000
D_IN = 256
D_HID = 128
N_CLASSES = 64

NC = 2
NS = 16
N_PAD = 10240
NPT = N_PAD // NS
CW = 100
ROWS = N_EDGES // CW
RPT = ROWS // (NC * NS)

_mesh = plsc.VectorSubcoreMesh(core_axis_name="c", subcore_axis_name="s")


def _deg_body(src_hbm, dst_hbm, out_hbm, sidx, didx, buf, acc_out, acc_in):
    c = lax.axis_index("c")
    s = lax.axis_index("s")
    for q in range(NPT // 16):
        buf[pl.ds(q * 16, 16)] = jnp.zeros((16,), jnp.float32)
    pltpu.sync_copy(buf, acc_out.at[pl.ds(s * NPT, NPT)])
    pltpu.sync_copy(buf, acc_in.at[pl.ds(s * NPT, NPT)])
    for q in range(CW // 16 + 1):
        buf[pl.ds(q * 16, 16)] = jnp.ones((16,), jnp.float32)
    plsc.subcore_barrier()

    wid = c * NS + s
    pltpu.sync_copy(src_hbm.at[wid], sidx)
    pltpu.sync_copy(dst_hbm.at[wid], didx)

    def body(j, carry):
        pltpu.sync_copy(buf.at[pl.ds(0, CW)], acc_out.at[sidx.at[j]], add=True)
        pltpu.sync_copy(buf.at[pl.ds(0, CW)], acc_in.at[didx.at[j]], add=True)
        return carry

    lax.fori_loop(0, RPT, body, 0)
    plsc.subcore_barrier()
    pltpu.sync_copy(acc_out.at[pl.ds(s * NPT, NPT)], out_hbm.at[c, 0, pl.ds(s * NPT, NPT)])
    pltpu.sync_copy(acc_in.at[pl.ds(s * NPT, NPT)], out_hbm.at[c, 1, pl.ds(s * NPT, NPT)])


_deg_call = pl.kernel(
    _deg_body,
    out_type=jax.ShapeDtypeStruct((NC, 2, N_PAD), jnp.float32),
    mesh=_mesh,
    scratch_types=[
        pltpu.VMEM((RPT, CW), jnp.int32),
        pltpu.VMEM((RPT, CW), jnp.int32),
        pltpu.VMEM((NPT,), jnp.float32),
        pltpu.VMEM_SHARED((N_PAD,), jnp.float32),
        pltpu.VMEM_SHARED((N_PAD,), jnp.float32),
    ],
)


def _make_agg(D):
    def _agg_body(y_hbm, src_hbm, dst_hbm, out_hbm, sidx, didx, rows_v, zbuf, acc, sem):
        c = lax.axis_index("c")
        s = lax.axis_index("s")
        for r in range(16):
            for q in range(D // 16):
                zbuf[r, pl.ds(q * 16, 16)] = jnp.zeros((16,), jnp.float32)

        def zloop(k, carry):
            pltpu.sync_copy(zbuf, acc.at[pl.ds(s * NPT + k * 16, 16), :])
            return carry

        lax.fori_loop(0, NPT // 16, zloop, 0)
        plsc.subcore_barrier()

        wid = c * NS + s
        pltpu.sync_copy(src_hbm.at[wid], sidx)
        pltpu.sync_copy(dst_hbm.at[wid], didx)

        def body(j, carry):
            pltpu.async_copy(y_hbm.at[sidx.at[j]], rows_v, sem).wait()
            pltpu.sync_copy(rows_v, acc.at[didx.at[j]], add=True)
            return carry

        lax.fori_loop(0, RPT, body, 0)
        plsc.subcore_barrier()
        pltpu.sync_copy(acc.at[pl.ds(s * NPT, NPT), :],
                        out_hbm.at[c, pl.ds(s * NPT, NPT), :])

    return pl.kernel(
        _agg_body,
        out_type=jax.ShapeDtypeStruct((NC, N_PAD, D), jnp.float32),
        mesh=_mesh,
        scratch_types=[
            pltpu.VMEM((RPT, CW), jnp.int32),
            pltpu.VMEM((RPT, CW), jnp.int32),
            pltpu.VMEM((CW, D), jnp.float32),
            pltpu.VMEM((16, D), jnp.float32),
            pltpu.VMEM_SHARED((N_PAD, D), jnp.float32),
            pltpu.SemaphoreType.DMA,
        ],
    )


_agg_hid = _make_agg(D_HID)


def _norm(d):
    return jnp.where(d > 0.0, lax.rsqrt(jnp.maximum(d, 1.0)), 0.0)


def _y1_body(x_ref, w_ref, dp_ref, o_ref):
    d_out = dp_ref[0, 0, :N_NODES] + dp_ref[1, 0, :N_NODES]
    ns = _norm(d_out)
    z = jnp.dot(x_ref[...], w_ref[...], preferred_element_type=jnp.float32)
    o_ref[...] = z * ns[:, None]


def _y2_body(a_ref, dp_ref, b1_ref, w_ref, o_ref):
    a = a_ref[0, :N_NODES, :] + a_ref[1, :N_NODES, :]
    nd = _norm(dp_ref[0, 1, :N_NODES] + dp_ref[1, 1, :N_NODES])
    ns = _norm(dp_ref[0, 0, :N_NODES] + dp_ref[1, 0, :N_NODES])
    h = jnp.maximum(a * nd[:, None] + b1_ref[...][None, :], 0.0)
    o_ref[...] = jnp.dot(h * ns[:, None], w_ref[...],
                         preferred_element_type=jnp.float32)


def _out_body(a_ref, dp_ref, b2_ref, o_ref):
    a = a_ref[0, :N_NODES, :N_CLASSES] + a_ref[1, :N_NODES, :N_CLASSES]
    nd = _norm(dp_ref[0, 1, :N_NODES] + dp_ref[1, 1, :N_NODES])
    o_ref[...] = a * nd[:, None] + b2_ref[...][None, :]


def _tc_call(body, out_shape):
    return pl.pallas_call(body, out_shape=jax.ShapeDtypeStruct(out_shape, jnp.float32))


@jax.jit
def kernel(features, edge_index, W1, b1, W2, b2):
    src = edge_index[0].astype(jnp.int32).reshape(NC * NS, RPT, CW)
    dst = edge_index[1].astype(jnp.int32).reshape(NC * NS, RPT, CW)

    W2p = jnp.zeros((D_HID, D_HID), jnp.float32).at[:, :N_CLASSES].set(W2)

    dp = _deg_call(src, dst)
    y1 = _tc_call(_y1_body, (N_NODES, D_HID))(features, W1, dp)
    p1 = _agg_hid(y1, src, dst)
    y2 = _tc_call(_y2_body, (N_NODES, D_HID))(p1, dp, b1, W2p)
    p2 = _agg_hid(y2, src, dst)
    out = _tc_call(_out_body, (N_NODES, N_CLASSES))(p2, dp, b2)
    return out


# speedup vs baseline: 8.7962x; 1.1767x over previous
import functools

import jax
import jax.numpy as jnp
from jax import lax
from jax.experimental import pallas as pl
from jax.experimental.pallas import tpu as pltpu
from jax.experimental.pallas import tpu_sc as plsc

N_NODES = 10000
N_EDGES = 160000
D_IN = 256
D_HID = 128
N_CLASSES = 64

NC = 2
NS = 16
N_PAD = 10240
NPT = N_PAD // NS
CW = 100
ROWS = N_EDGES // CW
RPT = ROWS // (NC * NS)

_mesh = plsc.VectorSubcoreMesh(core_axis_name="c", subcore_axis_name="s")


def _deg_body(src_hbm, dst_hbm, out_hbm, sidx, didx, buf, acc_out, acc_in):
    c = lax.axis_index("c")
    s = lax.axis_index("s")
    for q in range(NPT // 16):
        buf[pl.ds(q * 16, 16)] = jnp.zeros((16,), jnp.float32)
    pltpu.sync_copy(buf, acc_out.at[pl.ds(s * NPT, NPT)])
    pltpu.sync_copy(buf, acc_in.at[pl.ds(s * NPT, NPT)])
    for q in range(CW // 16 + 1):
        buf[pl.ds(q * 16, 16)] = jnp.ones((16,), jnp.float32)
    plsc.subcore_barrier()

    wid = c * NS + s
    pltpu.sync_copy(src_hbm.at[wid], sidx)
    pltpu.sync_copy(dst_hbm.at[wid], didx)

    def body(j, carry):
        pltpu.sync_copy(buf.at[pl.ds(0, CW)], acc_out.at[sidx.at[j]], add=True)
        pltpu.sync_copy(buf.at[pl.ds(0, CW)], acc_in.at[didx.at[j]], add=True)
        return carry

    lax.fori_loop(0, RPT, body, 0)
    plsc.subcore_barrier()
    pltpu.sync_copy(acc_out.at[pl.ds(s * NPT, NPT)], out_hbm.at[c, 0, pl.ds(s * NPT, NPT)])
    pltpu.sync_copy(acc_in.at[pl.ds(s * NPT, NPT)], out_hbm.at[c, 1, pl.ds(s * NPT, NPT)])


_deg_call = pl.kernel(
    _deg_body,
    out_type=jax.ShapeDtypeStruct((NC, 2, N_PAD), jnp.float32),
    mesh=_mesh,
    scratch_types=[
        pltpu.VMEM((RPT, CW), jnp.int32),
        pltpu.VMEM((RPT, CW), jnp.int32),
        pltpu.VMEM((NPT,), jnp.float32),
        pltpu.VMEM_SHARED((N_PAD,), jnp.float32),
        pltpu.VMEM_SHARED((N_PAD,), jnp.float32),
    ],
)


NB = 2
NG = RPT // NB


def _make_agg(D):
    def _agg_body(y_hbm, src_hbm, dst_hbm, out_hbm, sidx, didx, zbuf, acc, *rest):
        bufs = rest[:NB]
        gsem = rest[NB:2 * NB]
        ssem = rest[2 * NB:3 * NB]
        c = lax.axis_index("c")
        s = lax.axis_index("s")
        for r in range(16):
            for q in range(D // 16):
                zbuf[r, pl.ds(q * 16, 16)] = jnp.zeros((16,), jnp.float32)

        def zloop(k, carry):
            pltpu.sync_copy(zbuf, acc.at[pl.ds(s * NPT + k * 16, 16), :])
            return carry

        lax.fori_loop(0, NPT // 16, zloop, 0)
        plsc.subcore_barrier()

        wid = c * NS + s
        pltpu.sync_copy(src_hbm.at[wid], sidx)
        pltpu.sync_copy(dst_hbm.at[wid], didx)

        def start_gather(b, j):
            pltpu.async_copy(y_hbm.at[sidx.at[j]], bufs[b], gsem[b])

        def wait_gather(b, j):
            pltpu.make_async_copy(y_hbm.at[sidx.at[j]], bufs[b], gsem[b]).wait()

        def start_scatter(b, j):
            pltpu.async_copy(bufs[b], acc.at[didx.at[j]], ssem[b], add=True)

        def wait_scatter(b, j):
            pltpu.make_async_copy(bufs[b], acc.at[didx.at[j]], ssem[b]).wait()

        for b in range(NB):
            start_gather(b, b)

        def gbody(g, carry):
            for b in range(NB):
                wait_gather(b, g * NB + b)
                start_scatter(b, g * NB + b)
            for b in range(NB):
                wait_scatter(b, g * NB + b)
                start_gather(b, (g + 1) * NB + b)
            return carry

        lax.fori_loop(0, NG - 1, gbody, 0)
        g_last = NG - 1
        for b in range(NB):
            wait_gather(b, g_last * NB + b)
            start_scatter(b, g_last * NB + b)
        for b in range(NB):
            wait_scatter(b, g_last * NB + b)

        plsc.subcore_barrier()
        pltpu.sync_copy(acc.at[pl.ds(s * NPT, NPT), :],
                        out_hbm.at[c, pl.ds(s * NPT, NPT), :])

    return pl.kernel(
        _agg_body,
        out_type=jax.ShapeDtypeStruct((NC, N_PAD, D), jnp.float32),
        mesh=_mesh,
        scratch_types=[
            pltpu.VMEM((RPT, CW), jnp.int32),
            pltpu.VMEM((RPT, CW), jnp.int32),
            pltpu.VMEM((16, D), jnp.float32),
            pltpu.VMEM_SHARED((N_PAD, D), jnp.float32),
        ] + [pltpu.VMEM((CW, D), jnp.float32) for _ in range(NB)]
          + [pltpu.SemaphoreType.DMA for _ in range(2 * NB)],
    )


_agg_hid = _make_agg(D_HID)


def _norm(d):
    return jnp.where(d > 0.0, lax.rsqrt(jnp.maximum(d, 1.0)), 0.0)


def _y1_body(x_ref, w_ref, dp_ref, o_ref):
    d_out = dp_ref[0, 0, :N_NODES] + dp_ref[1, 0, :N_NODES]
    ns = _norm(d_out)
    z = jnp.dot(x_ref[...], w_ref[...], preferred_element_type=jnp.float32)
    o_ref[...] = z * ns[:, None]


def _y2_body(a_ref, dp_ref, b1_ref, w_ref, o_ref):
    a = a_ref[0, :N_NODES, :] + a_ref[1, :N_NODES, :]
    nd = _norm(dp_ref[0, 1, :N_NODES] + dp_ref[1, 1, :N_NODES])
    ns = _norm(dp_ref[0, 0, :N_NODES] + dp_ref[1, 0, :N_NODES])
    h = jnp.maximum(a * nd[:, None] + b1_ref[...][None, :], 0.0)
    o_ref[...] = jnp.dot(h * ns[:, None], w_ref[...],
                         preferred_element_type=jnp.float32)


def _out_body(a_ref, dp_ref, b2_ref, o_ref):
    a = a_ref[0, :N_NODES, :N_CLASSES] + a_ref[1, :N_NODES, :N_CLASSES]
    nd = _norm(dp_ref[0, 1, :N_NODES] + dp_ref[1, 1, :N_NODES])
    o_ref[...] = a * nd[:, None] + b2_ref[...][None, :]


def _tc_call(body, out_shape):
    return pl.pallas_call(body, out_shape=jax.ShapeDtypeStruct(out_shape, jnp.float32))


@jax.jit
def kernel(features, edge_index, W1, b1, W2, b2):
    src = edge_index[0].astype(jnp.int32).reshape(NC * NS, RPT, CW)
    dst = edge_index[1].astype(jnp.int32).reshape(NC * NS, RPT, CW)

    W2p = jnp.zeros((D_HID, D_HID), jnp.float32).at[:, :N_CLASSES].set(W2)

    dp = _deg_call(src, dst)
    y1 = _tc_call(_y1_body, (N_NODES, D_HID))(features, W1, dp)
    p1 = _agg_hid(y1, src, dst)
    y2 = _tc_call(_y2_body, (N_NODES, D_HID))(p1, dp, b1, W2p)
    p2 = _agg_hid(y2, src, dst)
    out = _tc_call(_out_body, (N_NODES, N_CLASSES))(p2, dp, b2)
    return out


# speedup vs baseline: 8.9828x; 1.0212x over previous
import functools

import jax
import jax.numpy as jnp
from jax import lax
from jax.experimental import pallas as pl
from jax.experimental.pallas import tpu as pltpu
from jax.experimental.pallas import tpu_sc as plsc

N_NODES = 10000
N_EDGES = 160000
D_IN = 256
D_HID = 128
N_CLASSES = 64

NC = 2
NS = 16
N_PAD = 10240
NPT = N_PAD // NS
CW = 125
ROWS = N_EDGES // CW
RPT = ROWS // (NC * NS)

_mesh = plsc.VectorSubcoreMesh(core_axis_name="c", subcore_axis_name="s")


def _deg_body(src_hbm, dst_hbm, out_hbm, sidx, didx, buf, acc_out, acc_in):
    c = lax.axis_index("c")
    s = lax.axis_index("s")
    for q in range(NPT // 16):
        buf[pl.ds(q * 16, 16)] = jnp.zeros((16,), jnp.float32)
    pltpu.sync_copy(buf, acc_out.at[pl.ds(s * NPT, NPT)])
    pltpu.sync_copy(buf, acc_in.at[pl.ds(s * NPT, NPT)])
    for q in range(CW // 16 + 1):
        buf[pl.ds(q * 16, 16)] = jnp.ones((16,), jnp.float32)
    plsc.subcore_barrier()

    wid = c * NS + s
    pltpu.sync_copy(src_hbm.at[wid], sidx)
    pltpu.sync_copy(dst_hbm.at[wid], didx)

    def body(j, carry):
        pltpu.sync_copy(buf.at[pl.ds(0, CW)], acc_out.at[sidx.at[j]], add=True)
        pltpu.sync_copy(buf.at[pl.ds(0, CW)], acc_in.at[didx.at[j]], add=True)
        return carry

    lax.fori_loop(0, RPT, body, 0)
    plsc.subcore_barrier()
    pltpu.sync_copy(acc_out.at[pl.ds(s * NPT, NPT)], out_hbm.at[c, 0, pl.ds(s * NPT, NPT)])
    pltpu.sync_copy(acc_in.at[pl.ds(s * NPT, NPT)], out_hbm.at[c, 1, pl.ds(s * NPT, NPT)])


_deg_call = pl.kernel(
    _deg_body,
    out_type=jax.ShapeDtypeStruct((NC, 2, N_PAD), jnp.float32),
    mesh=_mesh,
    scratch_types=[
        pltpu.VMEM((RPT, CW), jnp.int32),
        pltpu.VMEM((RPT, CW), jnp.int32),
        pltpu.VMEM((NPT,), jnp.float32),
        pltpu.VMEM_SHARED((N_PAD,), jnp.float32),
        pltpu.VMEM_SHARED((N_PAD,), jnp.float32),
    ],
)


NB = 2
NG = RPT // NB


def _make_agg(D):
    def _agg_body(y_hbm, src_hbm, dst_hbm, out_hbm, sidx, didx, zbuf, acc, *rest):
        bufs = rest[:NB]
        gsem = rest[NB:2 * NB]
        ssem = rest[2 * NB:3 * NB]
        c = lax.axis_index("c")
        s = lax.axis_index("s")
        for r in range(16):
            for q in range(D // 16):
                zbuf[r, pl.ds(q * 16, 16)] = jnp.zeros((16,), jnp.float32)

        def zloop(k, carry):
            pltpu.sync_copy(zbuf, acc.at[pl.ds(s * NPT + k * 16, 16), :])
            return carry

        lax.fori_loop(0, NPT // 16, zloop, 0)
        plsc.subcore_barrier()

        wid = c * NS + s
        pltpu.sync_copy(src_hbm.at[wid], sidx)
        pltpu.sync_copy(dst_hbm.at[wid], didx)

        def start_gather(b, j):
            pltpu.async_copy(y_hbm.at[sidx.at[j]], bufs[b], gsem[b])

        def wait_gather(b, j):
            pltpu.make_async_copy(y_hbm.at[sidx.at[j]], bufs[b], gsem[b]).wait()

        def start_scatter(b, j):
            pltpu.async_copy(bufs[b], acc.at[didx.at[j]], ssem[b], add=True)

        def wait_scatter(b, j):
            pltpu.make_async_copy(bufs[b], acc.at[didx.at[j]], ssem[b]).wait()

        for b in range(NB):
            start_gather(b, b)

        def gbody(g, carry):
            for b in range(NB):
                wait_gather(b, g * NB + b)
                start_scatter(b, g * NB + b)
            for b in range(NB):
                wait_scatter(b, g * NB + b)
                start_gather(b, (g + 1) * NB + b)
            return carry

        lax.fori_loop(0, NG - 1, gbody, 0)
        g_last = NG - 1
        for b in range(NB):
            wait_gather(b, g_last * NB + b)
            start_scatter(b, g_last * NB + b)
        for b in range(NB):
            wait_scatter(b, g_last * NB + b)

        plsc.subcore_barrier()
        pltpu.sync_copy(acc.at[pl.ds(s * NPT, NPT), :],
                        out_hbm.at[c, pl.ds(s * NPT, NPT), :])

    return pl.kernel(
        _agg_body,
        out_type=jax.ShapeDtypeStruct((NC, N_PAD, D), jnp.float32),
        mesh=_mesh,
        scratch_types=[
            pltpu.VMEM((RPT, CW), jnp.int32),
            pltpu.VMEM((RPT, CW), jnp.int32),
            pltpu.VMEM((16, D), jnp.float32),
            pltpu.VMEM_SHARED((N_PAD, D), jnp.float32),
        ] + [pltpu.VMEM((CW, D), jnp.float32) for _ in range(NB)]
          + [pltpu.SemaphoreType.DMA for _ in range(2 * NB)],
    )


_agg_hid = _make_agg(D_HID)


def _norm(d):
    return jnp.where(d > 0.0, lax.rsqrt(jnp.maximum(d, 1.0)), 0.0)


def _y1_body(x_ref, w_ref, dp_ref, o_ref):
    d_out = dp_ref[0, 0, :N_NODES] + dp_ref[1, 0, :N_NODES]
    ns = _norm(d_out)
    z = jnp.dot(x_ref[...], w_ref[...], preferred_element_type=jnp.float32)
    o_ref[...] = z * ns[:, None]


def _y2_body(a_ref, dp_ref, b1_ref, w_ref, o_ref):
    a = a_ref[0, :N_NODES, :] + a_ref[1, :N_NODES, :]
    nd = _norm(dp_ref[0, 1, :N_NODES] + dp_ref[1, 1, :N_NODES])
    ns = _norm(dp_ref[0, 0, :N_NODES] + dp_ref[1, 0, :N_NODES])
    h = jnp.maximum(a * nd[:, None] + b1_ref[...][None, :], 0.0)
    o_ref[...] = jnp.dot(h * ns[:, None], w_ref[...],
                         preferred_element_type=jnp.float32)


def _out_body(a_ref, dp_ref, b2_ref, o_ref):
    a = a_ref[0, :N_NODES, :N_CLASSES] + a_ref[1, :N_NODES, :N_CLASSES]
    nd = _norm(dp_ref[0, 1, :N_NODES] + dp_ref[1, 1, :N_NODES])
    o_ref[...] = a * nd[:, None] + b2_ref[...][None, :]


def _tc_call(body, out_shape):
    return pl.pallas_call(body, out_shape=jax.ShapeDtypeStruct(out_shape, jnp.float32))


@jax.jit
def kernel(features, edge_index, W1, b1, W2, b2):
    src = edge_index[0].astype(jnp.int32).reshape(NC * NS, RPT, CW)
    dst = edge_index[1].astype(jnp.int32).reshape(NC * NS, RPT, CW)

    W2p = jnp.zeros((D_HID, D_HID), jnp.float32).at[:, :N_CLASSES].set(W2)

    dp = _deg_call(src, dst)
    y1 = _tc_call(_y1_body, (N_NODES, D_HID))(features, W1, dp)
    p1 = _agg_hid(y1, src, dst)
    y2 = _tc_call(_y2_body, (N_NODES, D_HID))(p1, dp, b1, W2p)
    p2 = _agg_hid(y2, src, dst)
    out = _tc_call(_out_body, (N_NODES, N_CLASSES))(p2, dp, b2)
    return out
